# Initial kernel scaffold; baseline (speedup 1.0000x reference)
import jax
import jax.numpy as jnp
from jax.experimental import pallas as pl


def kernel(x_ent, x_event, x_center, h_ent, h_event, h_center, e2c_src, e2c_dst, ee_src, ee_dst, Wf, bf, Wg):
    raise NotImplementedError("write your pallas kernel here")


# speedup vs baseline: 1.5169x; 1.5169x over previous
import dataclasses
import functools

import jax
import jax.numpy as jnp
from jax import lax
from jax.experimental import pallas as pl
from jax.experimental.pallas import tpu as pltpu
from jax.experimental.pallas import tpu_sc as plsc

N_ENT = 20000
N_EVENT = 20000
N_CENTER = 2000
E_E2C = 40000
E_EE = 60000
D = 768

NC = 2
NS = 16

E2C_PAD = 40960
EE_PAD = 61440
C_PAD = 2048
EV_PAD = 20480

CW = 128
NCH = D // CW
HALF = EV_PAD // 2
JUNK = HALF
ACC_ROWS = HALF + 128

NB_E2C = E2C_PAD // 128 // NS
NB_EE = EE_PAD // 128 // NS

BM = 1000

_SC_MESH = plsc.VectorSubcoreMesh(
    core_axis_name="c", subcore_axis_name="s", num_cores=NC, num_subcores=NS)

_SC_PARAMS = pltpu.CompilerParams()
if "needs_layout_passes" in pltpu.CompilerParams.__dataclass_fields__:
    _SC_PARAMS = dataclasses.replace(_SC_PARAMS, needs_layout_passes=False)


def _fill2d(ref, value):
    rows, cols = ref.shape
    v = jnp.full((16,), value, jnp.float32)

    @pl.loop(0, rows)
    def _(i):
        @pl.loop(0, cols, step=16)
        def _(j):
            ref[i, pl.ds(j, 16)] = v


def _fill1d(ref, value):
    (n,) = ref.shape
    v = jnp.full((16,), value, jnp.float32)

    @pl.loop(0, n, step=16)
    def _(i):
        ref[pl.ds(i, 16)] = v


def _center_agg_body(tbl_hbm, out_hbm, src_v, dst_v, rows_v, acc, sem,
                     core, sub):
    for ch in range(NCH // 2):
        col = (core * (NCH // 2) + ch) * CW

        _fill2d(rows_v, 0.0)
        pltpu.sync_copy(rows_v, acc.at[pl.ds(sub * 128, 128)])
        plsc.subcore_barrier()

        @pl.loop(0, NB_E2C)
        def _(j):
            pltpu.async_copy(
                tbl_hbm.at[src_v.at[j], pl.ds(col, CW)], rows_v, sem).wait()
            pltpu.sync_copy(rows_v, acc.at[dst_v.at[j]], add=True)

        plsc.subcore_barrier()
        pltpu.sync_copy(acc.at[pl.ds(sub * 128, 128)],
                        out_hbm.at[pl.ds(sub * 128, 128), pl.ds(col, CW)])


def _sc_init(x_event, e2c_src2, e2c_dst2, ee_dst2):
    @functools.partial(
        pl.kernel,
        out_type=(
            jax.ShapeDtypeStruct((C_PAD, D), jnp.float32),
            jax.ShapeDtypeStruct((NS, C_PAD), jnp.float32),
            jax.ShapeDtypeStruct((NS, EV_PAD), jnp.float32),
        ),
        mesh=_SC_MESH,
        scratch_types=[
            pltpu.VMEM((NB_E2C, 128), jnp.int32),
            pltpu.VMEM((NB_E2C, 128), jnp.int32),
            pltpu.VMEM((NB_EE, 128), jnp.int32),
            pltpu.VMEM((128, CW), jnp.float32),
            pltpu.VMEM((C_PAD,), jnp.float32),
            pltpu.VMEM((EV_PAD,), jnp.float32),
            pltpu.VMEM_SHARED((C_PAD, CW), jnp.float32),
            pltpu.SemaphoreType.DMA,
        ],
        compiler_params=_SC_PARAMS,
    )
    def k(xev_hbm, src_hbm, dst_hbm, eed_hbm, c0_out, degc_out, dege_out,
          src_v, dst_v, eed_v, rows_v, hc_v, he_v, acc, sem):
        core = lax.axis_index("c")
        sub = lax.axis_index("s")
        ones = jnp.ones((16,), jnp.float32)

        pltpu.sync_copy(src_hbm.at[sub], src_v)
        pltpu.sync_copy(dst_hbm.at[sub], dst_v)

        @pl.when(core == 0)
        def _():
            _fill1d(hc_v, 0.0)

            @pl.loop(0, NB_E2C)
            def _(j):
                @pl.loop(0, 128, step=16)
                def _(g):
                    plsc.addupdate_scatter(hc_v, [dst_v[j, pl.ds(g, 16)]], ones)

            pltpu.sync_copy(hc_v, degc_out.at[sub])

        @pl.when(core == 1)
        def _():
            _fill1d(he_v, 0.0)
            pltpu.sync_copy(eed_hbm.at[sub], eed_v)

            @pl.loop(0, NB_EE)
            def _(j):
                @pl.loop(0, 128, step=16)
                def _(g):
                    plsc.addupdate_scatter(he_v, [eed_v[j, pl.ds(g, 16)]], ones)

            pltpu.sync_copy(he_v, dege_out.at[sub])

        _center_agg_body(xev_hbm, c0_out, src_v, dst_v, rows_v, acc, sem,
                         core, sub)

    return k(x_event, e2c_src2, e2c_dst2, ee_dst2)


def _sc_mc(table, src2, dst2):
    @functools.partial(
        pl.kernel,
        out_type=jax.ShapeDtypeStruct((C_PAD, D), jnp.float32),
        mesh=_SC_MESH,
        scratch_types=[
            pltpu.VMEM((NB_E2C, 128), jnp.int32),
            pltpu.VMEM((NB_E2C, 128), jnp.int32),
            pltpu.VMEM((128, CW), jnp.float32),
            pltpu.VMEM_SHARED((C_PAD, CW), jnp.float32),
            pltpu.SemaphoreType.DMA,
        ],
        compiler_params=_SC_PARAMS,
    )
    def k(tbl_hbm, src_hbm, dst_hbm, out_hbm, src_v, dst_v, rows_v, acc, sem):
        core = lax.axis_index("c")
        sub = lax.axis_index("s")
        pltpu.sync_copy(src_hbm.at[sub], src_v)
        pltpu.sync_copy(dst_hbm.at[sub], dst_v)
        _center_agg_body(tbl_hbm, out_hbm, src_v, dst_v, rows_v, acc, sem,
                         core, sub)

    return k(table, src2, dst2)


def _sc_mev(table, src2, dst2):
    stripe = HALF // NS

    @functools.partial(
        pl.kernel,
        out_type=jax.ShapeDtypeStruct((EV_PAD, D), jnp.float32),
        mesh=_SC_MESH,
        scratch_types=[
            pltpu.VMEM((NB_EE, 128), jnp.int32),
            pltpu.VMEM((NB_EE, 128), jnp.int32),
            pltpu.VMEM((NB_EE, 128), jnp.int32),
            pltpu.VMEM((128, CW), jnp.float32),
            pltpu.VMEM((128, CW), jnp.float32),
            pltpu.VMEM_SHARED((ACC_ROWS, CW), jnp.float32),
            pltpu.SemaphoreType.DMA,
        ],
        compiler_params=_SC_PARAMS,
    )
    def k(tbl_hbm, src_hbm, dst_hbm, out_hbm,
          src_v, dst_v, dtr_v, rows_v, z_v, acc, sem):
        core = lax.axis_index("c")
        sub = lax.axis_index("s")

        _fill2d(z_v, 0.0)
        pltpu.sync_copy(src_hbm.at[sub], src_v)
        pltpu.sync_copy(dst_hbm.at[sub], dst_v)

        for half in range(2):
            lo = half * HALF

            @pl.loop(0, NB_EE)
            def _(j):
                @pl.loop(0, 128, step=16)
                def _(g):
                    d = dst_v[j, pl.ds(g, 16)] - lo
                    ok = (d >= 0) & (d < HALF)
                    dtr_v[j, pl.ds(g, 16)] = jnp.where(ok, d, JUNK)

            for ch in range(NCH // 2):
                col = (core * (NCH // 2) + ch) * CW

                @pl.loop(0, stripe // 128)
                def _(i):
                    pltpu.sync_copy(
                        z_v, acc.at[pl.ds(sub * stripe + i * 128, 128)])

                plsc.subcore_barrier()

                @pl.loop(0, NB_EE)
                def _(j):
                    pltpu.async_copy(
                        tbl_hbm.at[src_v.at[j], pl.ds(col, CW)],
                        rows_v, sem).wait()
                    pltpu.sync_copy(rows_v, acc.at[dtr_v.at[j]], add=True)

                plsc.subcore_barrier()

                @pl.loop(0, stripe // 128)
                def _(i):
                    r = sub * stripe + i * 128
                    pltpu.sync_copy(
                        acc.at[pl.ds(r, 128)],
                        out_hbm.at[pl.ds(lo + r, 128), pl.ds(col, CW)])

    return k(table, src2, dst2)


def _dot(a, b):
    return jnp.dot(a, b, preferred_element_type=jnp.float32)


def _tc_fuse(hf, h, wt, wb, b):
    m = hf.shape[0]
    bm = min(BM, m)

    def body(hf_ref, h_ref, wt_ref, wb_ref, b_ref, o_ref):
        acc = _dot(hf_ref[...], wt_ref[...]) + _dot(h_ref[...], wb_ref[...])
        o_ref[...] = acc + b_ref[...]

    return pl.pallas_call(
        body,
        grid=(m // bm,),
        in_specs=[
            pl.BlockSpec((bm, D), lambda i: (i, 0)),
            pl.BlockSpec((bm, D), lambda i: (i, 0)),
            pl.BlockSpec((D, D), lambda i: (0, 0)),
            pl.BlockSpec((D, D), lambda i: (0, 0)),
            pl.BlockSpec((1, D), lambda i: (0, 0)),
        ],
        out_specs=pl.BlockSpec((bm, D), lambda i: (i, 0)),
        out_shape=jax.ShapeDtypeStruct((m, D), jnp.float32),
    )(hf, h, wt, wb, b.reshape(1, D))


def _tc_fuse_center0(hf, c0s, degcp, wt, wb, b):
    def body(hf_ref, p_ref, d_ref, wt_ref, wb_ref, b_ref, o_ref):
        deg = jnp.maximum(jnp.sum(d_ref[:, :N_CENTER], axis=0), 1.0)
        c0 = p_ref[:N_CENTER, :] / deg[:, None]
        acc = _dot(hf_ref[...], wt_ref[...]) + _dot(c0, wb_ref[...])
        o_ref[...] = acc + b_ref[...]

    return pl.pallas_call(
        body,
        out_shape=jax.ShapeDtypeStruct((N_CENTER, D), jnp.float32),
    )(hf, c0s, degcp, wt, wb, b.reshape(1, D))


def _tc_proj_ent(h, wg):
    def body(h_ref, w_ref, o_ref):
        o_ref[...] = jnp.maximum(_dot(h_ref[...], w_ref[...]), 0.0)

    return pl.pallas_call(
        body,
        grid=(N_ENT // BM,),
        in_specs=[pl.BlockSpec((BM, D), lambda i: (i, 0)),
                  pl.BlockSpec((D, D), lambda i: (0, 0))],
        out_shape=jax.ShapeDtypeStruct((N_ENT, D), jnp.float32),
        out_specs=pl.BlockSpec((BM, D), lambda i: (i, 0)),
    )(h, wg)


def _tc_proj_event(h, msum, degep, wg):
    bm = 1024

    def body(h_ref, m_ref, d_ref, w_ref, o_ref):
        dblk = d_ref[:, pl.ds(pl.program_id(0) * bm, bm)]
        deg = jnp.maximum(jnp.sum(dblk, axis=0), 1.0)
        x = h_ref[...] + m_ref[...] / deg[:, None]
        o_ref[...] = jnp.maximum(_dot(x, w_ref[...]), 0.0)

    return pl.pallas_call(
        body,
        grid=(EV_PAD // bm,),
        in_specs=[pl.BlockSpec((bm, D), lambda i: (i, 0)),
                  pl.BlockSpec((bm, D), lambda i: (i, 0)),
                  pl.BlockSpec((NS, EV_PAD), lambda i: (0, 0)),
                  pl.BlockSpec((D, D), lambda i: (0, 0))],
        out_shape=jax.ShapeDtypeStruct((N_EVENT, D), jnp.float32),
        out_specs=pl.BlockSpec((bm, D), lambda i: (i, 0)),
    )(h, msum, degep, wg)


def _tc_proj_center(h, mcs, degcp, wg):
    def body(h_ref, p_ref, d_ref, w_ref, o_ref):
        deg = jnp.maximum(jnp.sum(d_ref[:, :N_CENTER], axis=0), 1.0)
        m = p_ref[:N_CENTER, :] / deg[:, None]
        o_ref[...] = jnp.maximum(_dot(h_ref[...] + m, w_ref[...]), 0.0)

    return pl.pallas_call(
        body,
        out_shape=jax.ShapeDtypeStruct((N_CENTER, D), jnp.float32),
    )(h, mcs, degcp, wg)


def _pad_idx(idx, total, fill):
    pad = jnp.full((total - idx.shape[0],), fill, jnp.int32)
    return jnp.concatenate([idx, pad]).reshape(NS, -1, 128)


def kernel(x_ent, x_event, x_center, h_ent, h_event, h_center,
           e2c_src, e2c_dst, ee_src, ee_dst, Wf, bf, Wg):
    e2c_src2 = _pad_idx(e2c_src, E2C_PAD, 0)
    e2c_dst2 = _pad_idx(e2c_dst, E2C_PAD, C_PAD - 1)
    ee_src2 = _pad_idx(ee_src, EE_PAD, 0)
    ee_dst2 = _pad_idx(ee_dst, EE_PAD, EV_PAD - 1)

    c0s, degcp, degep = _sc_init(x_event, e2c_src2, e2c_dst2, ee_dst2)

    hf = {"ent": h_ent, "event": h_event, "center": h_center}
    h = {"ent": x_ent, "event": x_event}

    for l in range(2):
        wt = {t: Wf[l, i, :D] for i, t in enumerate(("ent", "event", "center"))}
        wb = {t: Wf[l, i, D:] for i, t in enumerate(("ent", "event", "center"))}
        bias = {t: bf[l, i] for i, t in enumerate(("ent", "event", "center"))}

        hent = _tc_fuse(hf["ent"], h["ent"], wt["ent"], wb["ent"], bias["ent"])
        hev = _tc_fuse(hf["event"], h["event"], wt["event"], wb["event"],
                       bias["event"])
        if l == 0:
            hc = _tc_fuse_center0(hf["center"], c0s, degcp,
                                  wt["center"], wb["center"], bias["center"])
        else:
            hc = _tc_fuse(hf["center"], h["center"], wt["center"],
                          wb["center"], bias["center"])

        msum = _sc_mev(hent, ee_src2, ee_dst2)
        mcs = _sc_mc(hev, e2c_src2, e2c_dst2)

        h = {
            "ent": _tc_proj_ent(hent, Wg[l, 0]),
            "event": _tc_proj_event(hev, msum, degep, Wg[l, 1]),
            "center": _tc_proj_center(hc, mcs, degcp, Wg[l, 2]),
        }

    return (h["ent"], h["event"], h["center"])


# speedup vs baseline: 1.7731x; 1.1689x over previous
import dataclasses
import functools

import jax
import jax.numpy as jnp
from jax import lax
from jax.experimental import pallas as pl
from jax.experimental.pallas import tpu as pltpu
from jax.experimental.pallas import tpu_sc as plsc

N_ENT = 20000
N_EVENT = 20000
N_CENTER = 2000
E_E2C = 40000
E_EE = 60000
D = 768

NC = 2
NS = 16

E2C_PAD = 40960
EE_PAD = 61440
C_PAD = 2048
EV_PAD = 20480

CW = 128
NCH = D // CW
HALF = EV_PAD // 2
JUNK = HALF
ACC_ROWS = HALF + 128

NB_E2C = E2C_PAD // 128 // NS
NB_EE = EE_PAD // 128 // NS

BM = 1000

_SC_MESH = plsc.VectorSubcoreMesh(
    core_axis_name="c", subcore_axis_name="s", num_cores=NC, num_subcores=NS)

_SC_PARAMS = pltpu.CompilerParams()
if "needs_layout_passes" in pltpu.CompilerParams.__dataclass_fields__:
    _SC_PARAMS = dataclasses.replace(_SC_PARAMS, needs_layout_passes=False)


def _fill2d(ref, value):
    rows, cols = ref.shape
    v = jnp.full((16,), value, jnp.float32)

    @pl.loop(0, rows)
    def _(i):
        @pl.loop(0, cols, step=16)
        def _(j):
            ref[i, pl.ds(j, 16)] = v


def _fill1d(ref, value):
    (n,) = ref.shape
    v = jnp.full((16,), value, jnp.float32)

    @pl.loop(0, n, step=16)
    def _(i):
        ref[pl.ds(i, 16)] = v


def _agg_pass(tbl_hbm, col, nb, src_v, dst_v, buf_a, buf_b, acc,
              sem_a, sem_b):
    def gather(j, buf, sem):
        pltpu.async_copy(tbl_hbm.at[src_v.at[j], pl.ds(col, CW)], buf, sem)

    def wait(buf, sem):
        pltpu.make_async_copy(
            tbl_hbm.at[pl.ds(0, 128), pl.ds(col, CW)], buf, sem).wait()

    def scat(j, buf):
        pltpu.sync_copy(buf, acc.at[dst_v.at[j]], add=True)

    gather(0, buf_a, sem_a)

    @pl.loop(0, nb - 2, step=2)
    def _(j):
        gather(j + 1, buf_b, sem_b)
        wait(buf_a, sem_a)
        scat(j, buf_a)
        gather(j + 2, buf_a, sem_a)
        wait(buf_b, sem_b)
        scat(j + 1, buf_b)

    gather(nb - 1, buf_b, sem_b)
    wait(buf_a, sem_a)
    scat(nb - 2, buf_a)
    wait(buf_b, sem_b)
    scat(nb - 1, buf_b)


def _center_agg_body(tbl_hbm, out_hbm, src_v, dst_v, buf_a, buf_b, acc,
                     sem_a, sem_b, core, sub):
    @pl.loop(0, NCH // 2)
    def _(ch):
        col = (core * (NCH // 2) + ch) * CW

        _fill2d(buf_a, 0.0)
        pltpu.sync_copy(buf_a, acc.at[pl.ds(sub * 128, 128)])
        plsc.subcore_barrier()

        _agg_pass(tbl_hbm, col, NB_E2C, src_v, dst_v, buf_a, buf_b, acc,
                  sem_a, sem_b)

        plsc.subcore_barrier()
        pltpu.sync_copy(acc.at[pl.ds(sub * 128, 128)],
                        out_hbm.at[pl.ds(sub * 128, 128), pl.ds(col, CW)])


def _sc_init(x_event, e2c_src2, e2c_dst2, ee_dst2):
    @functools.partial(
        pl.kernel,
        out_type=(
            jax.ShapeDtypeStruct((C_PAD, D), jnp.float32),
            jax.ShapeDtypeStruct((NS, C_PAD), jnp.float32),
            jax.ShapeDtypeStruct((NS, EV_PAD), jnp.float32),
        ),
        mesh=_SC_MESH,
        scratch_types=[
            pltpu.VMEM((NB_E2C, 128), jnp.int32),
            pltpu.VMEM((NB_E2C, 128), jnp.int32),
            pltpu.VMEM((NB_EE, 128), jnp.int32),
            pltpu.VMEM((128, CW), jnp.float32),
            pltpu.VMEM((128, CW), jnp.float32),
            pltpu.VMEM((C_PAD,), jnp.float32),
            pltpu.VMEM((EV_PAD,), jnp.float32),
            pltpu.VMEM_SHARED((C_PAD, CW), jnp.float32),
            pltpu.SemaphoreType.DMA,
            pltpu.SemaphoreType.DMA,
        ],
        compiler_params=_SC_PARAMS,
    )
    def k(xev_hbm, src_hbm, dst_hbm, eed_hbm, c0_out, degc_out, dege_out,
          src_v, dst_v, eed_v, buf_a, buf_b, hc_v, he_v, acc, sem_a, sem_b):
        core = lax.axis_index("c")
        sub = lax.axis_index("s")
        ones = jnp.ones((16,), jnp.float32)

        pltpu.sync_copy(src_hbm.at[sub], src_v)
        pltpu.sync_copy(dst_hbm.at[sub], dst_v)

        @pl.when(core == 0)
        def _():
            _fill1d(hc_v, 0.0)

            @pl.loop(0, NB_E2C)
            def _(j):
                @pl.loop(0, 128, step=16)
                def _(g):
                    plsc.addupdate_scatter(hc_v, [dst_v[j, pl.ds(g, 16)]], ones)

            pltpu.sync_copy(hc_v, degc_out.at[sub])

        @pl.when(core == 1)
        def _():
            _fill1d(he_v, 0.0)
            pltpu.sync_copy(eed_hbm.at[sub], eed_v)

            @pl.loop(0, NB_EE)
            def _(j):
                @pl.loop(0, 128, step=16)
                def _(g):
                    plsc.addupdate_scatter(he_v, [eed_v[j, pl.ds(g, 16)]], ones)

            pltpu.sync_copy(he_v, dege_out.at[sub])

        _center_agg_body(xev_hbm, c0_out, src_v, dst_v, buf_a, buf_b, acc,
                         sem_a, sem_b, core, sub)

    return k(x_event, e2c_src2, e2c_dst2, ee_dst2)


def _sc_mc(table, src2, dst2):
    @functools.partial(
        pl.kernel,
        out_type=jax.ShapeDtypeStruct((C_PAD, D), jnp.float32),
        mesh=_SC_MESH,
        scratch_types=[
            pltpu.VMEM((NB_E2C, 128), jnp.int32),
            pltpu.VMEM((NB_E2C, 128), jnp.int32),
            pltpu.VMEM((128, CW), jnp.float32),
            pltpu.VMEM((128, CW), jnp.float32),
            pltpu.VMEM_SHARED((C_PAD, CW), jnp.float32),
            pltpu.SemaphoreType.DMA,
            pltpu.SemaphoreType.DMA,
        ],
        compiler_params=_SC_PARAMS,
    )
    def k(tbl_hbm, src_hbm, dst_hbm, out_hbm, src_v, dst_v, buf_a, buf_b,
          acc, sem_a, sem_b):
        core = lax.axis_index("c")
        sub = lax.axis_index("s")
        pltpu.sync_copy(src_hbm.at[sub], src_v)
        pltpu.sync_copy(dst_hbm.at[sub], dst_v)
        _center_agg_body(tbl_hbm, out_hbm, src_v, dst_v, buf_a, buf_b, acc,
                         sem_a, sem_b, core, sub)

    return k(table, src2, dst2)


def _sc_mev(table, src2, dst2):
    stripe = HALF // NS

    @functools.partial(
        pl.kernel,
        out_type=jax.ShapeDtypeStruct((EV_PAD, D), jnp.float32),
        mesh=_SC_MESH,
        scratch_types=[
            pltpu.VMEM((NB_EE, 128), jnp.int32),
            pltpu.VMEM((NB_EE, 128), jnp.int32),
            pltpu.VMEM((NB_EE, 128), jnp.int32),
            pltpu.VMEM((128, CW), jnp.float32),
            pltpu.VMEM((128, CW), jnp.float32),
            pltpu.VMEM_SHARED((ACC_ROWS, CW), jnp.float32),
            pltpu.SemaphoreType.DMA,
            pltpu.SemaphoreType.DMA,
        ],
        compiler_params=_SC_PARAMS,
    )
    def k(tbl_hbm, src_hbm, dst_hbm, out_hbm,
          src_v, dst_v, dtr_v, buf_a, buf_b, acc, sem_a, sem_b):
        core = lax.axis_index("c")
        sub = lax.axis_index("s")

        pltpu.sync_copy(src_hbm.at[sub], src_v)
        pltpu.sync_copy(dst_hbm.at[sub], dst_v)

        @pl.loop(0, 2)
        def _(half):
            lo = half * HALF

            @pl.loop(0, NB_EE)
            def _(j):
                @pl.loop(0, 128, step=16)
                def _(g):
                    d = dst_v[j, pl.ds(g, 16)] - lo
                    ok = (d >= 0) & (d < HALF)
                    dtr_v[j, pl.ds(g, 16)] = jnp.where(ok, d, JUNK)

            @pl.loop(0, NCH // 2)
            def _(ch):
                col = (core * (NCH // 2) + ch) * CW

                _fill2d(buf_a, 0.0)

                @pl.loop(0, stripe // 128)
                def _(i):
                    pltpu.sync_copy(
                        buf_a, acc.at[pl.ds(sub * stripe + i * 128, 128)])

                plsc.subcore_barrier()

                _agg_pass(tbl_hbm, col, NB_EE, src_v, dtr_v, buf_a, buf_b,
                          acc, sem_a, sem_b)

                plsc.subcore_barrier()

                @pl.loop(0, stripe // 128)
                def _(i):
                    r = sub * stripe + i * 128
                    pltpu.sync_copy(
                        acc.at[pl.ds(r, 128)],
                        out_hbm.at[pl.ds(lo + r, 128), pl.ds(col, CW)])

    return k(table, src2, dst2)


def _dot(a, b):
    return jnp.dot(a, b, preferred_element_type=jnp.float32)


def _tc_fuse(hf, h, wt, wb, b):
    m = hf.shape[0]
    bm = min(BM, m)

    def body(hf_ref, h_ref, wt_ref, wb_ref, b_ref, o_ref):
        acc = _dot(hf_ref[...], wt_ref[...]) + _dot(h_ref[...], wb_ref[...])
        o_ref[...] = acc + b_ref[...]

    return pl.pallas_call(
        body,
        grid=(m // bm,),
        in_specs=[
            pl.BlockSpec((bm, D), lambda i: (i, 0)),
            pl.BlockSpec((bm, D), lambda i: (i, 0)),
            pl.BlockSpec((D, D), lambda i: (0, 0)),
            pl.BlockSpec((D, D), lambda i: (0, 0)),
            pl.BlockSpec((1, D), lambda i: (0, 0)),
        ],
        out_specs=pl.BlockSpec((bm, D), lambda i: (i, 0)),
        out_shape=jax.ShapeDtypeStruct((m, D), jnp.float32),
    )(hf, h, wt, wb, b.reshape(1, D))


def _tc_fuse_center0(hf, c0s, degcp, wt, wb, b):
    def body(hf_ref, p_ref, d_ref, wt_ref, wb_ref, b_ref, o_ref):
        deg = jnp.maximum(jnp.sum(d_ref[:, :N_CENTER], axis=0), 1.0)
        c0 = p_ref[:N_CENTER, :] / deg[:, None]
        acc = _dot(hf_ref[...], wt_ref[...]) + _dot(c0, wb_ref[...])
        o_ref[...] = acc + b_ref[...]

    return pl.pallas_call(
        body,
        out_shape=jax.ShapeDtypeStruct((N_CENTER, D), jnp.float32),
    )(hf, c0s, degcp, wt, wb, b.reshape(1, D))


def _tc_proj_ent(h, wg):
    def body(h_ref, w_ref, o_ref):
        o_ref[...] = jnp.maximum(_dot(h_ref[...], w_ref[...]), 0.0)

    return pl.pallas_call(
        body,
        grid=(N_ENT // BM,),
        in_specs=[pl.BlockSpec((BM, D), lambda i: (i, 0)),
                  pl.BlockSpec((D, D), lambda i: (0, 0))],
        out_shape=jax.ShapeDtypeStruct((N_ENT, D), jnp.float32),
        out_specs=pl.BlockSpec((BM, D), lambda i: (i, 0)),
    )(h, wg)


def _tc_proj_event(h, msum, degep, wg):
    bm = 1024

    def body(h_ref, m_ref, d_ref, w_ref, o_ref):
        dblk = d_ref[:, pl.ds(pl.program_id(0) * bm, bm)]
        deg = jnp.maximum(jnp.sum(dblk, axis=0), 1.0)
        x = h_ref[...] + m_ref[...] / deg[:, None]
        o_ref[...] = jnp.maximum(_dot(x, w_ref[...]), 0.0)

    return pl.pallas_call(
        body,
        grid=(EV_PAD // bm,),
        in_specs=[pl.BlockSpec((bm, D), lambda i: (i, 0)),
                  pl.BlockSpec((bm, D), lambda i: (i, 0)),
                  pl.BlockSpec((NS, EV_PAD), lambda i: (0, 0)),
                  pl.BlockSpec((D, D), lambda i: (0, 0))],
        out_shape=jax.ShapeDtypeStruct((N_EVENT, D), jnp.float32),
        out_specs=pl.BlockSpec((bm, D), lambda i: (i, 0)),
    )(h, msum, degep, wg)


def _tc_proj_center(h, mcs, degcp, wg):
    def body(h_ref, p_ref, d_ref, w_ref, o_ref):
        deg = jnp.maximum(jnp.sum(d_ref[:, :N_CENTER], axis=0), 1.0)
        m = p_ref[:N_CENTER, :] / deg[:, None]
        o_ref[...] = jnp.maximum(_dot(h_ref[...] + m, w_ref[...]), 0.0)

    return pl.pallas_call(
        body,
        out_shape=jax.ShapeDtypeStruct((N_CENTER, D), jnp.float32),
    )(h, mcs, degcp, wg)


def _pad_idx(idx, total, fill):
    pad = jnp.full((total - idx.shape[0],), fill, jnp.int32)
    return jnp.concatenate([idx, pad]).reshape(NS, -1, 128)


def kernel(x_ent, x_event, x_center, h_ent, h_event, h_center,
           e2c_src, e2c_dst, ee_src, ee_dst, Wf, bf, Wg):
    e2c_src2 = _pad_idx(e2c_src, E2C_PAD, 0)
    e2c_dst2 = _pad_idx(e2c_dst, E2C_PAD, C_PAD - 1)
    ee_src2 = _pad_idx(ee_src, EE_PAD, 0)
    ee_dst2 = _pad_idx(ee_dst, EE_PAD, EV_PAD - 1)

    c0s, degcp, degep = _sc_init(x_event, e2c_src2, e2c_dst2, ee_dst2)

    hf = {"ent": h_ent, "event": h_event, "center": h_center}
    h = {"ent": x_ent, "event": x_event}

    for l in range(2):
        wt = {t: Wf[l, i, :D] for i, t in enumerate(("ent", "event", "center"))}
        wb = {t: Wf[l, i, D:] for i, t in enumerate(("ent", "event", "center"))}
        bias = {t: bf[l, i] for i, t in enumerate(("ent", "event", "center"))}

        hent = _tc_fuse(hf["ent"], h["ent"], wt["ent"], wb["ent"], bias["ent"])
        hev = _tc_fuse(hf["event"], h["event"], wt["event"], wb["event"],
                       bias["event"])
        if l == 0:
            hc = _tc_fuse_center0(hf["center"], c0s, degcp,
                                  wt["center"], wb["center"], bias["center"])
        else:
            hc = _tc_fuse(hf["center"], h["center"], wt["center"],
                          wb["center"], bias["center"])

        msum = _sc_mev(hent, ee_src2, ee_dst2)
        mcs = _sc_mc(hev, e2c_src2, e2c_dst2)

        h = {
            "ent": _tc_proj_ent(hent, Wg[l, 0]),
            "event": _tc_proj_event(hev, msum, degep, Wg[l, 1]),
            "center": _tc_proj_center(hc, mcs, degcp, Wg[l, 2]),
        }

    return (h["ent"], h["event"], h["center"])
